# baseline (speedup 1.0000x reference)
import math

import jax
import jax.numpy as jnp
from jax import lax
from jax.experimental import pallas as pl
from jax.experimental.pallas import tpu as pltpu
from jax.experimental.pallas import tpu_sc as plsc

_NC = 2
_NS = 16
_NW = _NC * _NS

_BATCH = 4
_SEQ = 2048
_D = 128
_TOKENS = _BATCH * _SEQ
_B_PER_W = _TOKENS // _NW
_CHUNKS = _B_PER_W // 128
_GROUP = _SEQ // _B_PER_W

_SCALE = 1.0 / math.sqrt(2.0 * math.log(_D))


_NCHUNK = 4
_CROWS = _B_PER_W // _NCHUNK


def _sc_gather_body(table_hbm, x_hbm, out_hbm, idx_v, rows_v,
                    g0, g1, g2, g3, wsem, isem):
    wid = lax.axis_index("s") * _NC + lax.axis_index("c")
    b = wid // _GROUP
    s0 = (wid % _GROUP) * _B_PER_W
    gsems = [g0, g1, g2, g3]
    idx_cps = [
        pltpu.async_copy(
            x_hbm.at[b, pl.ds(s0 + j * _CROWS, _CROWS)], idx_v.at[j], isem)
        for j in range(_NCHUNK)
    ]
    for c in idx_cps:
        c.wait()

    def gather(j):
        return pltpu.async_copy(
            table_hbm.at[idx_v.at[j]],
            rows_v.at[pl.ds(j * _CROWS, _CROWS)],
            gsems[j],
        )

    def write(j):
        return pltpu.async_copy(
            rows_v.at[pl.ds(j * _CROWS, _CROWS)],
            out_hbm.at[b, pl.ds(s0 + j * _CROWS, _CROWS)],
            wsem,
        )

    gcps = [gather(0), gather(1)]
    wcps = []
    for j in range(_NCHUNK):
        gcps[j].wait()
        wcps.append(write(j))
        if j + 2 < _NCHUNK:
            gcps.append(gather(j + 2))
    for c in wcps:
        c.wait()


def _sc_gather(word_table, x):
    mesh = plsc.VectorSubcoreMesh(
        core_axis_name="c", subcore_axis_name="s",
        num_cores=_NC, num_subcores=_NS,
    )
    return pl.kernel(
        _sc_gather_body,
        out_type=jax.ShapeDtypeStruct((_BATCH, _SEQ, _D), jnp.float32),
        mesh=mesh,
        scratch_types=[
            pltpu.VMEM((_NCHUNK, _CROWS), jnp.int32),
            pltpu.VMEM((_B_PER_W, _D), jnp.float32),
            pltpu.SemaphoreType.DMA,
            pltpu.SemaphoreType.DMA,
            pltpu.SemaphoreType.DMA,
            pltpu.SemaphoreType.DMA,
            pltpu.SemaphoreType.DMA,
            pltpu.SemaphoreType.DMA,
        ],
    )(word_table, x)


def _tc_norm_body(gw_ref, seg_ref, pos_ref, type_ref, gamma_ref, beta_ref,
                  out_ref):
    gw = gw_ref[0]
    segf = seg_ref[pl.program_id(0)].astype(jnp.float32)
    pos = pos_ref[...]
    t0 = type_ref[0:1, :]
    t1 = type_ref[1:2, :]
    gamma = gamma_ref[0:1, :]
    beta = beta_ref[0:1, :]

    emb = gw + pos + t0 + segf[:, None] * (t1 - t0)
    mean = jnp.mean(emb, axis=-1, keepdims=True)
    y = emb - mean
    xmin = jnp.min(y, axis=0, keepdims=True)
    xmax = jnp.max(y, axis=0, keepdims=True)
    inv = gamma / ((xmax - xmin) * _SCALE)
    out_ref[0] = y * inv + beta


def _tc_norm(gathered, seg, pos_table, type_table, gamma, beta):
    return pl.pallas_call(
        _tc_norm_body,
        grid=(_BATCH,),
        in_specs=[
            pl.BlockSpec((1, _SEQ, _D), lambda b: (b, 0, 0)),
            pl.BlockSpec((_BATCH, _SEQ), lambda b: (0, 0)),
            pl.BlockSpec((_SEQ, _D), lambda b: (0, 0)),
            pl.BlockSpec((2, _D), lambda b: (0, 0)),
            pl.BlockSpec((1, _D), lambda b: (0, 0)),
            pl.BlockSpec((1, _D), lambda b: (0, 0)),
        ],
        out_specs=pl.BlockSpec((1, _SEQ, _D), lambda b: (b, 0, 0)),
        out_shape=jax.ShapeDtypeStruct((_BATCH, _SEQ, _D), jnp.float32),
    )(gathered, seg, pos_table, type_table,
      gamma.reshape(1, _D), beta.reshape(1, _D))


def kernel(x, seg, word_table, pos_table, type_table, gamma, beta):
    if x.dtype != jnp.int32:
        x = x.astype(jnp.int32)
    if seg.dtype != jnp.int32:
        seg = seg.astype(jnp.int32)
    gathered = _sc_gather(word_table, x)
    return _tc_norm(gathered, seg, pos_table, type_table, gamma, beta)


# speedup vs baseline: 1.0279x; 1.0279x over previous
import math

import jax
import jax.numpy as jnp
from jax import lax
from jax.experimental import pallas as pl
from jax.experimental.pallas import tpu as pltpu
from jax.experimental.pallas import tpu_sc as plsc

_NC = 2
_NS = 16
_NW = _NC * _NS

_BATCH = 4
_SEQ = 2048
_D = 128
_TOKENS = _BATCH * _SEQ
_B_PER_W = _TOKENS // _NW
_CHUNKS = _B_PER_W // 128
_GROUP = _SEQ // _B_PER_W

_SCALE = 1.0 / math.sqrt(2.0 * math.log(_D))


def _sc_gather_body(table_hbm, x_hbm, out_hbm, idx_v, rows_v, sem):
    wid = lax.axis_index("s") * _NC + lax.axis_index("c")
    b = wid // _GROUP
    s0 = (wid % _GROUP) * _B_PER_W
    idx_cps = [
        pltpu.async_copy(
            x_hbm.at[b, pl.ds(s0 + j * 128, 128)], idx_v.at[j], sem)
        for j in range(_CHUNKS)
    ]
    for c in idx_cps:
        c.wait()
    copies = []
    for j in range(_CHUNKS):
        copies.append(
            pltpu.async_copy(
                table_hbm.at[idx_v.at[j]],
                rows_v.at[pl.ds(j * 128, 128)],
                sem,
            )
        )
    for c in copies:
        c.wait()
    pltpu.sync_copy(rows_v, out_hbm.at[b, pl.ds(s0, _B_PER_W)])


def _sc_gather(word_table, x):
    mesh = plsc.VectorSubcoreMesh(
        core_axis_name="c", subcore_axis_name="s",
        num_cores=_NC, num_subcores=_NS,
    )
    return pl.kernel(
        _sc_gather_body,
        out_type=jax.ShapeDtypeStruct((_BATCH, _SEQ, _D), jnp.float32),
        mesh=mesh,
        scratch_types=[
            pltpu.VMEM((_CHUNKS, 128), jnp.int32),
            pltpu.VMEM((_B_PER_W, _D), jnp.float32),
            pltpu.SemaphoreType.DMA,
        ],
    )(word_table, x)


def _tc_norm_body(gw_ref, seg_ref, pos_ref, type_ref, gamma_ref, beta_ref,
                  out_ref):
    gw = gw_ref[0]
    segf = seg_ref[pl.program_id(0)].astype(jnp.float32)
    pos = pos_ref[...]
    t0 = type_ref[0:1, :]
    t1 = type_ref[1:2, :]
    gamma = gamma_ref[0:1, :]
    beta = beta_ref[0:1, :]

    emb = gw + pos + t0 + segf[:, None] * (t1 - t0)
    mean = jnp.mean(emb, axis=-1, keepdims=True)
    y = emb - mean
    xmin = jnp.min(y, axis=0, keepdims=True)
    xmax = jnp.max(y, axis=0, keepdims=True)
    inv = gamma / ((xmax - xmin) * _SCALE)
    out_ref[0] = y * inv + beta


def _tc_norm(gathered, seg, pos_table, type_table, gamma, beta):
    return pl.pallas_call(
        _tc_norm_body,
        grid=(_BATCH,),
        in_specs=[
            pl.BlockSpec((1, _SEQ, _D), lambda b: (b, 0, 0)),
            pl.BlockSpec((_BATCH, _SEQ), lambda b: (0, 0)),
            pl.BlockSpec((_SEQ, _D), lambda b: (0, 0)),
            pl.BlockSpec((2, _D), lambda b: (0, 0)),
            pl.BlockSpec((1, _D), lambda b: (0, 0)),
            pl.BlockSpec((1, _D), lambda b: (0, 0)),
        ],
        out_specs=pl.BlockSpec((1, _SEQ, _D), lambda b: (b, 0, 0)),
        out_shape=jax.ShapeDtypeStruct((_BATCH, _SEQ, _D), jnp.float32),
    )(gathered, seg, pos_table, type_table,
      gamma.reshape(1, _D), beta.reshape(1, _D))


def kernel(x, seg, word_table, pos_table, type_table, gamma, beta):
    if x.dtype != jnp.int32:
        x = x.astype(jnp.int32)
    if seg.dtype != jnp.int32:
        seg = seg.astype(jnp.int32)
    gathered = _sc_gather(word_table, x)
    return _tc_norm(gathered, seg, pos_table, type_table, gamma, beta)
